# baseline (speedup 1.0000x reference)
import functools

import jax
import jax.numpy as jnp
from jax import lax
from jax.experimental import pallas as pl
from jax.experimental.pallas import tpu as pltpu
from jax.experimental.pallas import tpu_sc as plsc

N = 10000
E = 320000
D = 128

_NC = 2
_NS = 16
_NW = _NC * _NS
_C = 80
_PER_W = E // _NW
_NCHUNK = _PER_W // _C
_N_PAD = 10240
_ROWS_PER_TILE = _N_PAD // _NS
_ZR = 128


def _bdot(x, w):
    return jnp.dot(x.astype(jnp.bfloat16), w.astype(jnp.bfloat16),
                   preferred_element_type=jnp.float32)


def _pe_all_kernel(pe_ref, xn_ref, w1_ref, b1_ref, w2_ref, b2_ref, out_ref):
    l = pl.program_id(0)
    h = jnp.maximum(_bdot(pe_ref[...], w1_ref[0]) + b1_ref[0], 0.0)
    o = _bdot(h, w2_ref[0]) + b2_ref[0]
    out_ref[0] = jnp.where(l == 0, o + xn_ref[...], o)


def _pe_all(PE_p, X_n, W1s, b1s, W2s, b2s):
    B = 1000
    return pl.pallas_call(
        _pe_all_kernel,
        grid=(3, N // B),
        in_specs=[
            pl.BlockSpec((B, 128), lambda l, i: (i, 0)),
            pl.BlockSpec((B, 128), lambda l, i: (i, 0)),
            pl.BlockSpec((1, 128, 128), lambda l, i: (l, 0, 0)),
            pl.BlockSpec((1, 1, 128), lambda l, i: (l, 0, 0)),
            pl.BlockSpec((1, 128, 128), lambda l, i: (l, 0, 0)),
            pl.BlockSpec((1, 1, 128), lambda l, i: (l, 0, 0)),
        ],
        out_specs=pl.BlockSpec((1, B, 128), lambda l, i: (l, i, 0)),
        out_shape=jax.ShapeDtypeStruct((3, N, 128), jnp.float32),
    )(PE_p, X_n, W1s, b1s, W2s, b2s)


def _combine_kernel_pe(h2_ref, s_ref, a_ref, w1_ref, b1_ref, w2_ref, b2_ref,
                       pe_ref, out_ref):
    z = a_ref[0, 0] * h2_ref[...] + s_ref[0] + s_ref[1]
    h = jnp.maximum(_bdot(z, w1_ref[...]) + b1_ref[...], 0.0)
    out_ref[...] = _bdot(h, w2_ref[...]) + b2_ref[...] + pe_ref[0]


def _combine_kernel_nope(h2_ref, s_ref, a_ref, w1_ref, b1_ref, w2_ref, b2_ref,
                         out_ref):
    z = a_ref[0, 0] * h2_ref[...] + s_ref[0] + s_ref[1]
    h = jnp.maximum(_bdot(z, w1_ref[...]) + b1_ref[...], 0.0)
    out_ref[...] = _bdot(h, w2_ref[...]) + b2_ref[...]


def _combine_mlp(h2, S, a_arr, W1, b1, W2, b2, pe_all, l_next):
    B = 1000
    specs = [
        pl.BlockSpec((B, 128), lambda i: (i, 0)),
        pl.BlockSpec((2, B, 128), lambda i: (0, i, 0)),
        pl.BlockSpec((1, 1), lambda i: (0, 0)),
        pl.BlockSpec((128, 128), lambda i: (0, 0)),
        pl.BlockSpec((1, 128), lambda i: (0, 0)),
        pl.BlockSpec((128, 128), lambda i: (0, 0)),
        pl.BlockSpec((1, 128), lambda i: (0, 0)),
    ]
    args = [h2, S, a_arr, W1, b1, W2, b2]
    if l_next is None:
        body = _combine_kernel_nope
    else:
        body = _combine_kernel_pe
        specs.append(pl.BlockSpec((1, B, 128), lambda i: (l_next, i, 0)))
        args.append(pe_all)
    return pl.pallas_call(
        body,
        grid=(N // B,),
        in_specs=specs,
        out_specs=pl.BlockSpec((B, 128), lambda i: (i, 0)),
        out_shape=jax.ShapeDtypeStruct((N, 128), jnp.float32),
    )(*args)


def _edge_sc(h2, pk, dstp, emb_p):
    mesh = plsc.VectorSubcoreMesh(core_axis_name="c", subcore_axis_name="s")

    @functools.partial(
        pl.kernel, mesh=mesh,
        out_type=jax.ShapeDtypeStruct((_NC, _N_PAD, D), jnp.float32),
        scratch_types=[
            pltpu.VMEM((2, _C), jnp.int32),
            pltpu.VMEM((_C,), jnp.int32),
            pltpu.VMEM((_C,), jnp.int32),
            pltpu.VMEM((_C,), jnp.int32),
            pltpu.VMEM((_C,), jnp.int32),
            pltpu.VMEM((_C,), jnp.int32),
            pltpu.VMEM((_C,), jnp.int32),
            pltpu.VMEM((_C, D), jnp.float32),
            pltpu.VMEM((_C, D), jnp.float32),
            pltpu.VMEM((24, D), jnp.float32),
            pltpu.VMEM_SHARED((_N_PAD, D), jnp.float32),
            pltpu.SemaphoreType.DMA,
            pltpu.SemaphoreType.DMA,
            pltpu.SemaphoreType.DMA,
            pltpu.SemaphoreType.DMA,
            pltpu.SemaphoreType.DMA,
            pltpu.SemaphoreType.DMA,
            pltpu.SemaphoreType.DMA,
            pltpu.SemaphoreType.DMA,
        ],
    )
    def k(h2_hbm, pk_hbm, dst_hbm, emb_hbm, out_hbm,
          pkv, dvt0, dvt1, sv0, sv1, av0, av1, rows0, rows1, emb_v, s_sh,
          iksem0, iksem1, dxsem0, dxsem1, gsem0, gsem1,
          ssem0, ssem1):
        c = lax.axis_index("c")
        s = lax.axis_index("s")
        wid = c * _NS + s
        base = pl.multiple_of(wid * _PER_W, 8)
        dvt = (dvt0, dvt1)
        sv = (sv0, sv1)
        av = (av0, av1)
        rows_pair = (rows0, rows1)
        iksem = (iksem0, iksem1)
        dxsem = (dxsem0, dxsem1)
        gsem = (gsem0, gsem1)
        ssem = (ssem0, ssem1)

        pltpu.sync_copy(emb_hbm, emb_v)

        def zrow(r, _):
            for q in range(D // 16):
                rows0[r, pl.ds(q * 16, 16)] = jnp.zeros((16,), jnp.float32)
            return 0
        lax.fori_loop(0, _C, zrow, 0)
        tile_r0 = s * _ROWS_PER_TILE
        for kk in range(_ROWS_PER_TILE // _C):
            pltpu.sync_copy(rows0, s_sh.at[pl.ds(tile_r0 + kk * _C, _C)])
        plsc.subcore_barrier()

        def idx_pk(j, b):
            return pltpu.async_copy(pk_hbm.at[pl.ds(base + j * _C, _C)],
                                    pkv.at[b], iksem[b])

        def ik_wait(j, b):
            pltpu.make_async_copy(pk_hbm.at[pl.ds(base + j * _C, _C)],
                                  pkv.at[b], iksem[b]).wait()

        def idx_dst(j, b):
            return pltpu.async_copy(dst_hbm.at[pl.ds(base + j * _C, _C)],
                                    dvt[b], dxsem[b])

        def dx_wait(j, b):
            pltpu.make_async_copy(dst_hbm.at[pl.ds(base + j * _C, _C)],
                                  dvt[b], dxsem[b]).wait()

        def unpack(b):
            for q in range(_C // 16):
                sl = pl.ds(q * 16, 16)
                pkq = pkv[b, sl]
                sv[b][sl] = pkq >> 5
                av[b][sl] = pkq & 31

        def gathers(b):
            pltpu.async_copy(h2_hbm.at[sv[b]], rows_pair[b], gsem[b])

        def gathers_wait(b):
            pltpu.make_async_copy(h2_hbm.at[sv[b]], rows_pair[b],
                                  gsem[b]).wait()

        def compute(b):
            rb = rows_pair[b]

            @plsc.parallel_loop(0, _C // 16)
            def _(g):
                av16 = av[b][pl.ds(g * 16, 16)]
                a_list = [av16[l] for l in range(16)]
                for q in range(D // 16):
                    sl = pl.ds(q * 16, 16)
                    for l in range(16):
                        e = g * 16 + l
                        rb[e, sl] = jnp.maximum(
                            rb[e, sl] + emb_v[a_list[l], sl], 0.0)

        def scatter_start(b):
            return pltpu.async_copy(rows_pair[b], s_sh.at[dvt[b]], ssem[b],
                                    add=True)

        def scatter_wait(b):
            pltpu.make_async_copy(rows_pair[b], s_sh.at[dvt[b]],
                                  ssem[b]).wait()

        idx_pk(0, 0)
        idx_pk(1, 1)
        idx_dst(0, 0)
        ik_wait(0, 0)
        unpack(0)
        gathers(0)

        def body(j, b, last):
            nb = 1 - b

            @pl.when(j >= 1)
            def _():
                scatter_wait(nb)

            @pl.when(j + 1 < _NCHUNK)
            def _():
                ik_wait(j + 1, nb)
                unpack(nb)
                gathers(nb)

            @pl.when(j + 2 < _NCHUNK)
            def _():
                idx_pk(j + 2, b)
            gathers_wait(b)
            compute(b)
            dx_wait(j, b)
            scatter_start(b)

            @pl.when(j + 1 < _NCHUNK)
            def _():
                idx_dst(j + 1, nb)

        @pl.loop(0, (_NCHUNK - 1) // 2)
        def _(jj):
            for i in range(2):
                body(jj * 2 + i, i, False)

        body(_NCHUNK - 1, (_NCHUNK - 1) % 2, True)
        scatter_wait((_NCHUNK - 1) % 2)
        plsc.subcore_barrier()

        for kk in range(_ROWS_PER_TILE // _C):
            r0 = tile_r0 + kk * _C
            pltpu.sync_copy(s_sh.at[pl.ds(r0, _C)], rows0)
            pltpu.sync_copy(rows0, out_hbm.at[c, pl.ds(r0, _C)])

    return k(h2, pk, dstp, emb_p)


def kernel(X_n, edge_index, edge_attr, PE, params):
    src = edge_index[0].astype(jnp.int32)
    dstp = edge_index[1].astype(jnp.int32)
    attr = edge_attr.astype(jnp.int32)
    pk = (src << 5) | attr

    PE_p = jnp.pad(PE, ((0, 0), (0, 128 - PE.shape[1])))
    W1s = jnp.stack([
        jnp.pad(p["pe"]["W1"], ((0, 128 - PE.shape[1]), (0, 0)))
        for p in params])
    b1s = jnp.stack([p["pe"]["b1"] for p in params]).reshape(3, 1, 128)
    W2s = jnp.stack([p["pe"]["W2"] for p in params])
    b2s = jnp.stack([p["pe"]["b2"] for p in params]).reshape(3, 1, 128)

    pe_all = _pe_all(PE_p, X_n, W1s, b1s, W2s, b2s)

    h2 = pe_all[0]
    for l, p in enumerate(params):
        emb_p = jnp.pad(p["emb"], ((0, 24 - p["emb"].shape[0]), (0, 0)))
        S = _edge_sc(h2, pk, dstp, emb_p)
        a_arr = (1.0 + p["eps"]).reshape(1, 1)
        l_next = l + 1 if l + 1 < len(params) else None
        h2 = _combine_mlp(h2, S, a_arr, p["mlp"]["W1"],
                          p["mlp"]["b1"].reshape(1, 128), p["mlp"]["W2"],
                          p["mlp"]["b2"].reshape(1, 128), pe_all, l_next)
    return h2


# speedup vs baseline: 2.1651x; 2.1651x over previous
import functools

import jax
import jax.numpy as jnp
from jax import lax
from jax.experimental import pallas as pl
from jax.experimental.pallas import tpu as pltpu
from jax.experimental.pallas import tpu_sc as plsc

N = 10000
E = 320000
D = 128

_NC = 2
_NS = 16
_NW = _NC * _NS
_C = 80
_PER_W = E // _NW
_NCHUNK = _PER_W // _C
_N_PAD = 10240
_ROWS_PER_TILE = _N_PAD // _NS
_ZR = 128


def _bdot(x, w):
    return jnp.dot(x.astype(jnp.bfloat16), w.astype(jnp.bfloat16),
                   preferred_element_type=jnp.float32)


def _pe_all_kernel(pe_ref, xn_ref, w1_ref, b1_ref, w2_ref, b2_ref, out_ref):
    l = pl.program_id(0)
    h = jnp.maximum(_bdot(pe_ref[...], w1_ref[0]) + b1_ref[0], 0.0)
    o = _bdot(h, w2_ref[0]) + b2_ref[0]
    out_ref[0] = jnp.where(l == 0, o + xn_ref[...], o)


def _pe_all(PE_p, X_n, W1s, b1s, W2s, b2s):
    B = 1000
    return pl.pallas_call(
        _pe_all_kernel,
        grid=(3, N // B),
        in_specs=[
            pl.BlockSpec((B, 128), lambda l, i: (i, 0)),
            pl.BlockSpec((B, 128), lambda l, i: (i, 0)),
            pl.BlockSpec((1, 128, 128), lambda l, i: (l, 0, 0)),
            pl.BlockSpec((1, 1, 128), lambda l, i: (l, 0, 0)),
            pl.BlockSpec((1, 128, 128), lambda l, i: (l, 0, 0)),
            pl.BlockSpec((1, 1, 128), lambda l, i: (l, 0, 0)),
        ],
        out_specs=pl.BlockSpec((1, B, 128), lambda l, i: (l, i, 0)),
        out_shape=jax.ShapeDtypeStruct((3, N, 128), jnp.float32),
    )(PE_p, X_n, W1s, b1s, W2s, b2s)


def _combine_kernel_pe(h2_ref, s_ref, a_ref, w1_ref, b1_ref, w2_ref, b2_ref,
                       pe_ref, out_ref):
    z = a_ref[0, 0] * h2_ref[...] + s_ref[0] + s_ref[1]
    h = jnp.maximum(_bdot(z, w1_ref[...]) + b1_ref[...], 0.0)
    out_ref[...] = _bdot(h, w2_ref[...]) + b2_ref[...] + pe_ref[0]


def _combine_kernel_nope(h2_ref, s_ref, a_ref, w1_ref, b1_ref, w2_ref, b2_ref,
                         out_ref):
    z = a_ref[0, 0] * h2_ref[...] + s_ref[0] + s_ref[1]
    h = jnp.maximum(_bdot(z, w1_ref[...]) + b1_ref[...], 0.0)
    out_ref[...] = _bdot(h, w2_ref[...]) + b2_ref[...]


def _combine_mlp(h2, S, a_arr, W1, b1, W2, b2, pe_all, l_next):
    B = 1000
    specs = [
        pl.BlockSpec((B, 128), lambda i: (i, 0)),
        pl.BlockSpec((2, B, 128), lambda i: (0, i, 0)),
        pl.BlockSpec((1, 1), lambda i: (0, 0)),
        pl.BlockSpec((128, 128), lambda i: (0, 0)),
        pl.BlockSpec((1, 128), lambda i: (0, 0)),
        pl.BlockSpec((128, 128), lambda i: (0, 0)),
        pl.BlockSpec((1, 128), lambda i: (0, 0)),
    ]
    args = [h2, S, a_arr, W1, b1, W2, b2]
    if l_next is None:
        body = _combine_kernel_nope
    else:
        body = _combine_kernel_pe
        specs.append(pl.BlockSpec((1, B, 128), lambda i: (l_next, i, 0)))
        args.append(pe_all)
    return pl.pallas_call(
        body,
        grid=(N // B,),
        in_specs=specs,
        out_specs=pl.BlockSpec((B, 128), lambda i: (i, 0)),
        out_shape=jax.ShapeDtypeStruct((N, 128), jnp.float32),
    )(*args)


def _edge_sc(h2, pk, dstp, emb_p):
    mesh = plsc.VectorSubcoreMesh(core_axis_name="c", subcore_axis_name="s")

    @functools.partial(
        pl.kernel, mesh=mesh,
        out_type=jax.ShapeDtypeStruct((_NC, _N_PAD, D), jnp.float32),
        scratch_types=[
            pltpu.VMEM((2, _C), jnp.int32),
            pltpu.VMEM((_C,), jnp.int32),
            pltpu.VMEM((_C,), jnp.int32),
            pltpu.VMEM((_C,), jnp.int32),
            pltpu.VMEM((_C,), jnp.int32),
            pltpu.VMEM((_C,), jnp.int32),
            pltpu.VMEM((_C,), jnp.int32),
            pltpu.VMEM((_C, D), jnp.float32),
            pltpu.VMEM((_C, D), jnp.float32),
            pltpu.VMEM((_C, D), jnp.float32),
            pltpu.VMEM((_C, D), jnp.float32),
            pltpu.VMEM_SHARED((24, D), jnp.float32),
            pltpu.VMEM_SHARED((_N_PAD, D), jnp.float32),
            pltpu.SemaphoreType.DMA,
            pltpu.SemaphoreType.DMA,
            pltpu.SemaphoreType.DMA,
            pltpu.SemaphoreType.DMA,
            pltpu.SemaphoreType.DMA,
            pltpu.SemaphoreType.DMA,
            pltpu.SemaphoreType.DMA,
            pltpu.SemaphoreType.DMA,
            pltpu.SemaphoreType.DMA,
            pltpu.SemaphoreType.DMA,
        ],
    )
    def k(h2_hbm, pk_hbm, dst_hbm, emb_hbm, out_hbm,
          pkv, dvt0, dvt1, sv0, sv1, av0, av1, rows0, rows1, erows0, erows1,
          s_emb, s_sh,
          iksem0, iksem1, dxsem0, dxsem1, gsem0, gsem1, esem0, esem1,
          ssem0, ssem1):
        c = lax.axis_index("c")
        s = lax.axis_index("s")
        wid = c * _NS + s
        base = pl.multiple_of(wid * _PER_W, 8)
        dvt = (dvt0, dvt1)
        sv = (sv0, sv1)
        av = (av0, av1)
        rows_pair = (rows0, rows1)
        erows_pair = (erows0, erows1)
        iksem = (iksem0, iksem1)
        dxsem = (dxsem0, dxsem1)
        gsem = (gsem0, gsem1)
        esem = (esem0, esem1)
        ssem = (ssem0, ssem1)

        @pl.when(s == 0)
        def _():
            pltpu.sync_copy(emb_hbm, s_emb)

        def zrow(r, _):
            for q in range(D // 16):
                rows0[r, pl.ds(q * 16, 16)] = jnp.zeros((16,), jnp.float32)
            return 0
        lax.fori_loop(0, _C, zrow, 0)
        tile_r0 = s * _ROWS_PER_TILE
        for kk in range(_ROWS_PER_TILE // _C):
            pltpu.sync_copy(rows0, s_sh.at[pl.ds(tile_r0 + kk * _C, _C)])
        plsc.subcore_barrier()

        def idx_pk(j, b):
            return pltpu.async_copy(pk_hbm.at[pl.ds(base + j * _C, _C)],
                                    pkv.at[b], iksem[b])

        def ik_wait(j, b):
            pltpu.make_async_copy(pk_hbm.at[pl.ds(base + j * _C, _C)],
                                  pkv.at[b], iksem[b]).wait()

        def idx_dst(j, b):
            return pltpu.async_copy(dst_hbm.at[pl.ds(base + j * _C, _C)],
                                    dvt[b], dxsem[b])

        def dx_wait(j, b):
            pltpu.make_async_copy(dst_hbm.at[pl.ds(base + j * _C, _C)],
                                  dvt[b], dxsem[b]).wait()

        def unpack(b):
            for q in range(_C // 16):
                sl = pl.ds(q * 16, 16)
                pkq = pkv[b, sl]
                sv[b][sl] = pkq >> 5
                av[b][sl] = pkq & 31

        def gathers(b):
            pltpu.async_copy(h2_hbm.at[sv[b]], rows_pair[b], gsem[b])
            pltpu.async_copy(s_emb.at[av[b]], erows_pair[b], esem[b])

        def gathers_wait(b):
            pltpu.make_async_copy(h2_hbm.at[sv[b]], rows_pair[b],
                                  gsem[b]).wait()
            pltpu.make_async_copy(s_emb.at[av[b]], erows_pair[b],
                                  esem[b]).wait()

        def compute(b):
            rb = rows_pair[b]
            eb = erows_pair[b]

            @plsc.parallel_loop(0, _C, unroll=4)
            def _(e):
                for q in range(D // 16):
                    sl = pl.ds(q * 16, 16)
                    rb[e, sl] = jnp.maximum(rb[e, sl] + eb[e, sl], 0.0)

        def scatter_start(b):
            return pltpu.async_copy(rows_pair[b], s_sh.at[dvt[b]], ssem[b],
                                    add=True)

        def scatter_wait(b):
            pltpu.make_async_copy(rows_pair[b], s_sh.at[dvt[b]],
                                  ssem[b]).wait()

        idx_pk(0, 0)
        idx_pk(1, 1)
        idx_dst(0, 0)
        ik_wait(0, 0)
        unpack(0)
        gathers(0)

        def body(j, b, last):
            nb = 1 - b

            @pl.when(j >= 1)
            def _():
                scatter_wait(nb)

            @pl.when(j + 1 < _NCHUNK)
            def _():
                ik_wait(j + 1, nb)
                unpack(nb)
                gathers(nb)

            @pl.when(j + 2 < _NCHUNK)
            def _():
                idx_pk(j + 2, b)
            gathers_wait(b)
            compute(b)
            dx_wait(j, b)
            scatter_start(b)

            @pl.when(j + 1 < _NCHUNK)
            def _():
                idx_dst(j + 1, nb)

        @pl.loop(0, (_NCHUNK - 1) // 2)
        def _(jj):
            for i in range(2):
                body(jj * 2 + i, i, False)

        body(_NCHUNK - 1, (_NCHUNK - 1) % 2, True)
        scatter_wait((_NCHUNK - 1) % 2)
        plsc.subcore_barrier()

        for kk in range(_ROWS_PER_TILE // _C):
            r0 = tile_r0 + kk * _C
            pltpu.sync_copy(s_sh.at[pl.ds(r0, _C)], rows0)
            pltpu.sync_copy(rows0, out_hbm.at[c, pl.ds(r0, _C)])

    return k(h2, pk, dstp, emb_p)


def kernel(X_n, edge_index, edge_attr, PE, params):
    src = edge_index[0].astype(jnp.int32)
    dstp = edge_index[1].astype(jnp.int32)
    attr = edge_attr.astype(jnp.int32)
    pk = (src << 5) | attr

    PE_p = jnp.pad(PE, ((0, 0), (0, 128 - PE.shape[1])))
    W1s = jnp.stack([
        jnp.pad(p["pe"]["W1"], ((0, 128 - PE.shape[1]), (0, 0)))
        for p in params])
    b1s = jnp.stack([p["pe"]["b1"] for p in params]).reshape(3, 1, 128)
    W2s = jnp.stack([p["pe"]["W2"] for p in params])
    b2s = jnp.stack([p["pe"]["b2"] for p in params]).reshape(3, 1, 128)

    pe_all = _pe_all(PE_p, X_n, W1s, b1s, W2s, b2s)

    h2 = pe_all[0]
    for l, p in enumerate(params):
        emb_p = jnp.pad(p["emb"], ((0, 24 - p["emb"].shape[0]), (0, 0)))
        S = _edge_sc(h2, pk, dstp, emb_p)
        a_arr = (1.0 + p["eps"]).reshape(1, 1)
        l_next = l + 1 if l + 1 < len(params) else None
        h2 = _combine_mlp(h2, S, a_arr, p["mlp"]["W1"],
                          p["mlp"]["b1"].reshape(1, 128), p["mlp"]["W2"],
                          p["mlp"]["b2"].reshape(1, 128), pe_all, l_next)
    return h2
